# baseline (speedup 1.0000x reference)
import jax
import jax.numpy as jnp
from jax import lax
from jax.experimental import pallas as pl
from jax.experimental.pallas import tpu as pltpu
from jax.experimental.pallas import tpu_sc as plsc

DIM = 2048
BATCH = 16384

_info = plsc.get_sparse_core_info()
_NC = _info.num_cores
_NS = _info.num_subcores
_NW = _NC * _NS
_L = _info.num_lanes

_ROWS_PER_W = BATCH // _NW
_R = 8
_CHUNKS = _ROWS_PER_W // _R
_G = DIM // _L


def _permute_body(x_hbm, idx_hbm, out_hbm, idx_v, in_v, out_v):
    wid = lax.axis_index("s") * _NC + lax.axis_index("c")
    base = wid * _ROWS_PER_W
    pltpu.sync_copy(idx_hbm, idx_v)

    def chunk_body(ci, carry):
        elem0 = (base + ci * _R) * DIM
        pltpu.sync_copy(x_hbm.at[pl.ds(elem0, _R * DIM)], in_v)

        def g_body(g, c2):
            col = idx_v[pl.ds(g * _L, _L)]
            for r in range(_R):
                vals = plsc.load_gather(in_v, [col + (r * DIM)])
                out_v[pl.ds(r * DIM + g * _L, _L)] = vals
            return c2

        lax.fori_loop(0, _G, g_body, 0)
        pltpu.sync_copy(out_v, out_hbm.at[pl.ds(elem0, _R * DIM)])
        return carry

    lax.fori_loop(0, _CHUNKS, chunk_body, 0)


def _permute(x_flat, idx):
    kfn = pl.kernel(
        _permute_body,
        out_type=jax.ShapeDtypeStruct((BATCH * DIM,), jnp.float32),
        mesh=plsc.VectorSubcoreMesh(core_axis_name="c", subcore_axis_name="s"),
        compiler_params=pltpu.CompilerParams(needs_layout_passes=False),
        scratch_types=[
            pltpu.VMEM((DIM,), jnp.int32),
            pltpu.VMEM((_R * DIM,), jnp.float32),
            pltpu.VMEM((_R * DIM,), jnp.float32),
        ],
    )
    return kfn(x_flat, idx)


def kernel(x, logdet, perm, inv_perm, reverse):
    idx = jnp.where(reverse, inv_perm, perm).astype(jnp.int32)
    out_flat = _permute(x.reshape(BATCH * DIM), idx)
    return (out_flat.reshape(BATCH, DIM), logdet)


# speedup vs baseline: 2.0329x; 2.0329x over previous
import jax
import jax.numpy as jnp
from jax import lax
from jax.experimental import pallas as pl
from jax.experimental.pallas import tpu as pltpu
from jax.experimental.pallas import tpu_sc as plsc

DIM = 2048
BATCH = 16384

_info = plsc.get_sparse_core_info()
_NC = _info.num_cores
_NS = _info.num_subcores
_NW = _NC * _NS
_L = _info.num_lanes

_ROWS_PER_W = BATCH // _NW
_R = 8
_CHUNKS = _ROWS_PER_W // _R
_G = DIM // _L
_CH = _R * DIM


def _permute_body(x_hbm, idx_hbm, out_hbm,
                  idx_v, in_v0, in_v1, out_v0, out_v1,
                  sem_in0, sem_in1, sem_out0, sem_out1):
    wid = lax.axis_index("s") * _NC + lax.axis_index("c")
    base_elem = wid * _ROWS_PER_W * DIM
    in_bufs = (in_v0, in_v1)
    out_bufs = (out_v0, out_v1)
    sems_in = (sem_in0, sem_in1)
    sems_out = (sem_out0, sem_out1)

    pltpu.sync_copy(idx_hbm, idx_v)

    def start_in(c, b):
        pltpu.make_async_copy(
            x_hbm.at[pl.ds(base_elem + c * _CH, _CH)], in_bufs[b], sems_in[b]
        ).start()

    def wait_in(b):
        pltpu.make_async_copy(
            x_hbm.at[pl.ds(base_elem, _CH)], in_bufs[b], sems_in[b]
        ).wait()

    def start_out(c, b):
        pltpu.make_async_copy(
            out_bufs[b], out_hbm.at[pl.ds(base_elem + c * _CH, _CH)], sems_out[b]
        ).start()

    def wait_out(b):
        pltpu.make_async_copy(
            out_bufs[b], out_hbm.at[pl.ds(base_elem, _CH)], sems_out[b]
        ).wait()

    start_in(0, 0)
    start_in(1, 1)

    def pair_body(p, carry):
        c0 = p * 2
        for b in range(2):
            c = c0 + b
            wait_in(b)

            @pl.when(p >= 1)
            def _():
                wait_out(b)

            in_b = in_bufs[b]
            out_b = out_bufs[b]

            @plsc.parallel_loop(0, _G, unroll=4)
            def g_body(g):
                col = idx_v[pl.ds(g * _L, _L)]
                for r in range(_R):
                    vals = plsc.load_gather(in_b, [col + (r * DIM)])
                    out_b[pl.ds(r * DIM + g * _L, _L)] = vals

            start_out(c, b)

            @pl.when(c + 2 < _CHUNKS)
            def _():
                start_in(c + 2, b)

        return carry

    lax.fori_loop(0, _CHUNKS // 2, pair_body, 0)
    wait_out(0)
    wait_out(1)


def _permute(x_flat, idx):
    kfn = pl.kernel(
        _permute_body,
        out_type=jax.ShapeDtypeStruct((BATCH * DIM,), jnp.float32),
        mesh=plsc.VectorSubcoreMesh(core_axis_name="c", subcore_axis_name="s"),
        compiler_params=pltpu.CompilerParams(needs_layout_passes=False),
        scratch_types=[
            pltpu.VMEM((DIM,), jnp.int32),
            pltpu.VMEM((_CH,), jnp.float32),
            pltpu.VMEM((_CH,), jnp.float32),
            pltpu.VMEM((_CH,), jnp.float32),
            pltpu.VMEM((_CH,), jnp.float32),
            pltpu.SemaphoreType.DMA,
            pltpu.SemaphoreType.DMA,
            pltpu.SemaphoreType.DMA,
            pltpu.SemaphoreType.DMA,
        ],
    )
    return kfn(x_flat, idx)


def kernel(x, logdet, perm, inv_perm, reverse):
    idx = jnp.where(reverse, inv_perm, perm).astype(jnp.int32)
    out_flat = _permute(x.reshape(BATCH * DIM), idx)
    return (out_flat.reshape(BATCH, DIM), logdet)


# speedup vs baseline: 5.9361x; 2.9201x over previous
import jax
import jax.numpy as jnp
from jax import lax
from jax.experimental import pallas as pl
from jax.experimental.pallas import tpu as pltpu
from jax.experimental.pallas import tpu_sc as plsc

DIM = 2048
BATCH = 16384

_info = plsc.get_sparse_core_info()
_NC = _info.num_cores
_NS = _info.num_subcores
_NW = _NC * _NS
_L = _info.num_lanes

_ROWS_PER_W = BATCH // _NW
_R = 8
_CHUNKS = _ROWS_PER_W // _R
_G = DIM // _L


def _permute_body(x_hbm, idx_hbm, out_hbm,
                  idx_v, in_v0, in_v1, out_v0, out_v1,
                  sem_in0, sem_in1, sem_out0, sem_out1):
    wid = lax.axis_index("s") * _NC + lax.axis_index("c")
    base_row = wid * _ROWS_PER_W
    in_bufs = (in_v0, in_v1)
    out_bufs = (out_v0, out_v1)
    sems_in = (sem_in0, sem_in1)
    sems_out = (sem_out0, sem_out1)

    pltpu.sync_copy(idx_hbm, idx_v)

    def start_in(c, b):
        pltpu.make_async_copy(
            x_hbm.at[pl.ds(base_row + c * _R, _R)], in_bufs[b], sems_in[b]
        ).start()

    def wait_in(b):
        pltpu.make_async_copy(
            x_hbm.at[pl.ds(base_row, _R)], in_bufs[b], sems_in[b]
        ).wait()

    def start_out(c, b):
        pltpu.make_async_copy(
            out_bufs[b], out_hbm.at[pl.ds(base_row + c * _R, _R)], sems_out[b]
        ).start()

    def wait_out(b):
        pltpu.make_async_copy(
            out_bufs[b], out_hbm.at[pl.ds(base_row, _R)], sems_out[b]
        ).wait()

    start_in(0, 0)
    start_in(1, 1)

    def pair_body(p, carry):
        c0 = p * 2
        for b in range(2):
            c = c0 + b
            wait_in(b)

            @pl.when(p >= 1)
            def _():
                wait_out(b)

            in_b = in_bufs[b]
            out_b = out_bufs[b]

            @plsc.parallel_loop(0, _G, unroll=4)
            def g_body(g):
                col = idx_v[pl.ds(g * _L, _L)]
                for r in range(_R):
                    row_idx = jnp.full((_L,), r, jnp.int32)
                    vals = plsc.load_gather(in_b, [row_idx, col])
                    out_b[r, pl.ds(g * _L, _L)] = vals

            start_out(c, b)

            @pl.when(c + 2 < _CHUNKS)
            def _():
                start_in(c + 2, b)

        return carry

    lax.fori_loop(0, _CHUNKS // 2, pair_body, 0)
    wait_out(0)
    wait_out(1)


def _permute(x, idx):
    kfn = pl.kernel(
        _permute_body,
        out_type=jax.ShapeDtypeStruct((BATCH, DIM), jnp.float32),
        mesh=plsc.VectorSubcoreMesh(core_axis_name="c", subcore_axis_name="s"),
        compiler_params=pltpu.CompilerParams(needs_layout_passes=False),
        scratch_types=[
            pltpu.VMEM((DIM,), jnp.int32),
            pltpu.VMEM((_R, DIM), jnp.float32),
            pltpu.VMEM((_R, DIM), jnp.float32),
            pltpu.VMEM((_R, DIM), jnp.float32),
            pltpu.VMEM((_R, DIM), jnp.float32),
            pltpu.SemaphoreType.DMA,
            pltpu.SemaphoreType.DMA,
            pltpu.SemaphoreType.DMA,
            pltpu.SemaphoreType.DMA,
        ],
    )
    return kfn(x, idx)


def kernel(x, logdet, perm, inv_perm, reverse):
    idx = jnp.where(reverse, inv_perm, perm).astype(jnp.int32)
    out = _permute(x, idx)
    return (out, logdet)


# speedup vs baseline: 5.9389x; 1.0005x over previous
import jax
import jax.numpy as jnp
from jax import lax
from jax.experimental import pallas as pl
from jax.experimental.pallas import tpu as pltpu
from jax.experimental.pallas import tpu_sc as plsc

DIM = 2048
BATCH = 16384

_info = plsc.get_sparse_core_info()
_NC = _info.num_cores
_NS = _info.num_subcores
_NW = _NC * _NS
_L = _info.num_lanes

_ROWS_PER_W = BATCH // _NW
_R = 8
_CHUNKS = _ROWS_PER_W // _R
_G = DIM // _L


def _permute_body(x_hbm, idx_hbm, out_hbm,
                  idx_v, in_v0, in_v1, out_v0, out_v1,
                  sem_in0, sem_in1, sem_out0, sem_out1):
    wid = lax.axis_index("s") * _NC + lax.axis_index("c")
    base_row = wid * _ROWS_PER_W
    in_bufs = (in_v0, in_v1)
    out_bufs = (out_v0, out_v1)
    sems_in = (sem_in0, sem_in1)
    sems_out = (sem_out0, sem_out1)

    pltpu.sync_copy(idx_hbm, idx_v)

    def start_in(c, b):
        pltpu.make_async_copy(
            x_hbm.at[pl.ds(base_row + c * _R, _R)], in_bufs[b], sems_in[b]
        ).start()

    def wait_in(b):
        pltpu.make_async_copy(
            x_hbm.at[pl.ds(base_row, _R)], in_bufs[b], sems_in[b]
        ).wait()

    def start_out(c, b):
        pltpu.make_async_copy(
            out_bufs[b], out_hbm.at[pl.ds(base_row + c * _R, _R)], sems_out[b]
        ).start()

    def wait_out(b):
        pltpu.make_async_copy(
            out_bufs[b], out_hbm.at[pl.ds(base_row, _R)], sems_out[b]
        ).wait()

    start_in(0, 0)
    start_in(1, 1)

    def pair_body(p, carry):
        c0 = p * 2
        for b in range(2):
            c = c0 + b
            wait_in(b)

            @pl.when(p >= 1)
            def _():
                wait_out(b)

            in_b = in_bufs[b]
            out_b = out_bufs[b]

            @plsc.parallel_loop(0, _G, unroll=8)
            def g_body(g):
                col = idx_v[pl.ds(g * _L, _L)]
                for r in range(_R):
                    row_idx = jnp.full((_L,), r, jnp.int32)
                    vals = plsc.load_gather(in_b, [row_idx, col])
                    out_b[r, pl.ds(g * _L, _L)] = vals

            start_out(c, b)

            @pl.when(c + 2 < _CHUNKS)
            def _():
                start_in(c + 2, b)

        return carry

    lax.fori_loop(0, _CHUNKS // 2, pair_body, 0)
    wait_out(0)
    wait_out(1)


def _permute(x, idx):
    kfn = pl.kernel(
        _permute_body,
        out_type=jax.ShapeDtypeStruct((BATCH, DIM), jnp.float32),
        mesh=plsc.VectorSubcoreMesh(core_axis_name="c", subcore_axis_name="s"),
        compiler_params=pltpu.CompilerParams(needs_layout_passes=False),
        scratch_types=[
            pltpu.VMEM((DIM,), jnp.int32),
            pltpu.VMEM((_R, DIM), jnp.float32),
            pltpu.VMEM((_R, DIM), jnp.float32),
            pltpu.VMEM((_R, DIM), jnp.float32),
            pltpu.VMEM((_R, DIM), jnp.float32),
            pltpu.SemaphoreType.DMA,
            pltpu.SemaphoreType.DMA,
            pltpu.SemaphoreType.DMA,
            pltpu.SemaphoreType.DMA,
        ],
    )
    return kfn(x, idx)


def kernel(x, logdet, perm, inv_perm, reverse):
    idx = jnp.where(reverse, inv_perm, perm).astype(jnp.int32)
    out = _permute(x, idx)
    return (out, logdet)


# speedup vs baseline: 6.0247x; 1.0144x over previous
import jax
import jax.numpy as jnp
from jax import lax
from jax.experimental import pallas as pl
from jax.experimental.pallas import tpu as pltpu
from jax.experimental.pallas import tpu_sc as plsc

DIM = 2048
BATCH = 16384

_info = plsc.get_sparse_core_info()
_NC = _info.num_cores
_NS = _info.num_subcores
_NW = _NC * _NS
_L = _info.num_lanes

_ROWS_PER_W = BATCH // _NW
_R = 8
_H = _R // 2
_CHUNKS = _ROWS_PER_W // _R
_G = DIM // _L
_NIN = 4
_NOUT = 2


def _permute_body(x_hbm, idx_hbm, out_hbm,
                  idx_v, in_v0, in_v1, in_v2, in_v3, out_v0, out_v1,
                  sem_i0, sem_i1, sem_i2, sem_i3, sem_o0, sem_o1):
    wid = lax.axis_index("s") * _NC + lax.axis_index("c")
    base_row = wid * _ROWS_PER_W
    in_bufs = (in_v0, in_v1, in_v2, in_v3)
    out_bufs = (out_v0, out_v1)
    sems_in = (sem_i0, sem_i1, sem_i2, sem_i3)
    sems_out = (sem_o0, sem_o1)

    def start_in(c, b):
        pltpu.make_async_copy(
            x_hbm.at[pl.ds(base_row + c * _R, _R)], in_bufs[b], sems_in[b]
        ).start()

    def wait_in(b):
        pltpu.make_async_copy(
            x_hbm.at[pl.ds(base_row, _R)], in_bufs[b], sems_in[b]
        ).wait()

    def start_out_half(c, b, half):
        pltpu.make_async_copy(
            out_bufs[b].at[pl.ds(half * _H, _H)],
            out_hbm.at[pl.ds(base_row + c * _R + half * _H, _H)],
            sems_out[b],
        ).start()

    def wait_out(b):
        pltpu.make_async_copy(
            out_bufs[b], out_hbm.at[pl.ds(base_row, _R)], sems_out[b]
        ).wait()

    for b in range(_NIN):
        start_in(b, b)
    pltpu.sync_copy(idx_hbm, idx_v)

    def quad_body(q, carry):
        c0 = q * _NIN
        for k in range(_NIN):
            c = c0 + k
            bi = k
            bo = k % _NOUT
            wait_in(bi)

            if k >= _NOUT:
                wait_out(bo)
            else:
                @pl.when(q >= 1)
                def _():
                    wait_out(bo)

            in_b = in_bufs[bi]
            out_b = out_bufs[bo]

            for half in range(2):
                @plsc.parallel_loop(0, _G, unroll=8)
                def g_body(g):
                    col = idx_v[pl.ds(g * _L, _L)]
                    for r in range(half * _H, (half + 1) * _H):
                        row_idx = jnp.full((_L,), r, jnp.int32)
                        vals = plsc.load_gather(in_b, [row_idx, col])
                        out_b[r, pl.ds(g * _L, _L)] = vals

                start_out_half(c, bo, half)

            @pl.when(q <= (_CHUNKS // _NIN) - 2)
            def _():
                start_in(c + _NIN, bi)

        return carry

    lax.fori_loop(0, _CHUNKS // _NIN, quad_body, 0)
    wait_out(0)
    wait_out(1)


def _permute(x, idx):
    kfn = pl.kernel(
        _permute_body,
        out_type=jax.ShapeDtypeStruct((BATCH, DIM), jnp.float32),
        mesh=plsc.VectorSubcoreMesh(core_axis_name="c", subcore_axis_name="s"),
        compiler_params=pltpu.CompilerParams(needs_layout_passes=False),
        scratch_types=[
            pltpu.VMEM((DIM,), jnp.int32),
            pltpu.VMEM((_R, DIM), jnp.float32),
            pltpu.VMEM((_R, DIM), jnp.float32),
            pltpu.VMEM((_R, DIM), jnp.float32),
            pltpu.VMEM((_R, DIM), jnp.float32),
            pltpu.VMEM((_R, DIM), jnp.float32),
            pltpu.VMEM((_R, DIM), jnp.float32),
            pltpu.SemaphoreType.DMA,
            pltpu.SemaphoreType.DMA,
            pltpu.SemaphoreType.DMA,
            pltpu.SemaphoreType.DMA,
            pltpu.SemaphoreType.DMA,
            pltpu.SemaphoreType.DMA,
        ],
    )
    return kfn(x, idx)


def kernel(x, logdet, perm, inv_perm, reverse):
    idx = jnp.where(reverse, inv_perm, perm).astype(jnp.int32)
    out = _permute(x, idx)
    return (out, logdet)


# speedup vs baseline: 6.0449x; 1.0034x over previous
import jax
import jax.numpy as jnp
from jax import lax
from jax.experimental import pallas as pl
from jax.experimental.pallas import tpu as pltpu
from jax.experimental.pallas import tpu_sc as plsc

DIM = 2048
BATCH = 16384

_info = plsc.get_sparse_core_info()
_NC = _info.num_cores
_NS = _info.num_subcores
_NW = _NC * _NS
_L = _info.num_lanes

_ROWS_PER_W = BATCH // _NW
_R = 8
_H = _R // 2
_CHUNKS = _ROWS_PER_W // _R
_G = DIM // _L
_NIN = 4
_NOUT = 2


def _permute_body(x_hbm, idx_hbm, out_hbm,
                  idx_v, in_v0, in_v1, in_v2, in_v3, out_v0, out_v1,
                  sem_i0, sem_i1, sem_i2, sem_i3, sem_o0, sem_o1):
    wid = lax.axis_index("s") * _NC + lax.axis_index("c")
    base_row = wid * _ROWS_PER_W
    in_bufs = (in_v0, in_v1, in_v2, in_v3)
    out_bufs = (out_v0, out_v1)
    sems_in = (sem_i0, sem_i1, sem_i2, sem_i3)
    sems_out = (sem_o0, sem_o1)

    def start_in(c, b):
        pltpu.make_async_copy(
            x_hbm.at[pl.ds(base_row + c * _R, _R)], in_bufs[b], sems_in[b]
        ).start()

    def wait_in(b):
        pltpu.make_async_copy(
            x_hbm.at[pl.ds(base_row, _R)], in_bufs[b], sems_in[b]
        ).wait()

    def start_out_half(c, b, half):
        pltpu.make_async_copy(
            out_bufs[b].at[pl.ds(half * _H, _H)],
            out_hbm.at[pl.ds(base_row + c * _R + half * _H, _H)],
            sems_out[b],
        ).start()

    def wait_out(b):
        pltpu.make_async_copy(
            out_bufs[b], out_hbm.at[pl.ds(base_row, _R)], sems_out[b]
        ).wait()

    for b in range(_NIN):
        start_in(b, b)
    pltpu.sync_copy(idx_hbm, idx_v)

    def quad_body(q, carry):
        c0 = q * _NIN
        for k in range(_NIN):
            c = c0 + k
            bi = k
            bo = k % _NOUT
            wait_in(bi)

            if k >= _NOUT:
                wait_out(bo)
            else:
                @pl.when(q >= 1)
                def _():
                    wait_out(bo)

            in_b = in_bufs[bi]
            out_b = out_bufs[bo]

            for half in range(2):
                @plsc.parallel_loop(0, _G, unroll=8)
                def g_body(g):
                    col = idx_v[pl.ds(g * _L, _L)]
                    for r in range(half * _H, (half + 1) * _H):
                        row_idx = jnp.full((_L,), r, jnp.int32)
                        vals = plsc.load_gather(in_b, [row_idx, col])
                        out_b[r, pl.ds(g * _L, _L)] = vals

                start_out_half(c, bo, half)

            @pl.when(q <= (_CHUNKS // _NIN) - 2)
            def _():
                start_in(c + _NIN, bi)

        return carry

    lax.fori_loop(0, _CHUNKS // _NIN, quad_body, 0)
    wait_out(0)
    wait_out(1)


def _permute(x, idx):
    kfn = pl.kernel(
        _permute_body,
        out_type=jax.ShapeDtypeStruct((BATCH, DIM), jnp.float32),
        mesh=plsc.VectorSubcoreMesh(core_axis_name="c", subcore_axis_name="s"),
        compiler_params=pltpu.CompilerParams(
            needs_layout_passes=False,
            disable_bounds_checks=True,
        ),
        scratch_types=[
            pltpu.VMEM((DIM,), jnp.int32),
            pltpu.VMEM((_R, DIM), jnp.float32),
            pltpu.VMEM((_R, DIM), jnp.float32),
            pltpu.VMEM((_R, DIM), jnp.float32),
            pltpu.VMEM((_R, DIM), jnp.float32),
            pltpu.VMEM((_R, DIM), jnp.float32),
            pltpu.VMEM((_R, DIM), jnp.float32),
            pltpu.SemaphoreType.DMA,
            pltpu.SemaphoreType.DMA,
            pltpu.SemaphoreType.DMA,
            pltpu.SemaphoreType.DMA,
            pltpu.SemaphoreType.DMA,
            pltpu.SemaphoreType.DMA,
        ],
    )
    return kfn(x, idx)


def kernel(x, logdet, perm, inv_perm, reverse):
    idx = jnp.where(reverse, inv_perm, perm).astype(jnp.int32)
    out = _permute(x, idx)
    return (out, logdet)
